# baseline (speedup 1.0000x reference)
import functools

import jax
import jax.numpy as jnp
from jax import lax
from jax.experimental import pallas as pl
from jax.experimental.pallas import tpu as pltpu
from jax.experimental.pallas import tpu_sc as plsc

_B, _S = 8, 4
_T = _B * _S
_H = 1024
_E = 64
_EP = 128
_FF_TEXT = 1024
_FF_VIS = 512
_SH = 2048
_NORM_MIN = 1e-12
_NEG = -1e30
_L = 16


def _dotT(a, b):
    return jax.lax.dot_general(a, b, (((1,), (1,)), ((), ())),
                               preferred_element_type=jnp.float32)


def _logits_body(x_ref, tw_ref, vw_ref, ttc_ref, ttr_ref, tb_ref, vb_ref,
                 rl_ref, lcat_ref, meta_ref):
    x = x_ref[...]
    lt = _dotT(x, tw_ref[...])
    lv = _dotT(x, vw_ref[...])
    rl_ref[...] = jnp.where(ttc_ref[...] != 0, lv, lt)
    lcat_ref[...] = jnp.zeros((_T, 2 * _EP), jnp.float32)
    lcat_ref[:, :_E] = lt
    lcat_ref[:, _EP:_EP + _E] = lv
    meta_ref[...] = jnp.zeros((2, _EP), jnp.float32)
    meta_ref[0:1, :_T] = ttr_ref[...].astype(jnp.float32)
    meta_ref[0:1, _T:_T + _E] = tb_ref[...]
    meta_ref[1:2, :_E] = vb_ref[...]


def _sc_routing_body(lcat_hbm, meta_hbm, comb_hbm, sched_hbm,
                     row0, row1, c0, c1, pr, meta, sh, big, schedv):
    c = lax.axis_index("c")
    s = lax.axis_index("s")
    lanes = lax.broadcasted_iota(jnp.int32, (_L,), 0)
    zero = jnp.zeros((_L,), jnp.float32)

    @pl.when(c == 0)
    def _():
        pltpu.sync_copy(meta_hbm, meta)
        pltpu.sync_copy(lcat_hbm.at[s], row0)
        pltpu.sync_copy(lcat_hbm.at[s + _L], row1)

        for crow in (c0, c1):
            for k in list(range(4, 8)) + list(range(12, 16)):
                crow[pl.ds(_L * k, _L)] = zero

        for r, row, crow in ((0, row0, c0), (1, row1, c1)):
            ttf = meta[0, pl.ds(_L * r, _L)]
            tt_t = jnp.sum(jnp.where(lanes == s, ttf, 0.0))
            mtf = jnp.where(tt_t == 0.0, 1.0, 0.0)
            mvf = 1.0 - mtf

            a = []
            b = []
            for k in range(4):
                a.append(row[pl.ds(_L * k, _L)] * mtf
                         + row[pl.ds(_EP + _L * k, _L)] * mvf)
                b.append(meta[0, pl.ds(_T + _L * k, _L)] * mtf
                         + meta[1, pl.ds(_L * k, _L)] * mvf)

            m = jnp.max(a[0])
            for k in range(1, 4):
                m = jnp.maximum(m, jnp.max(a[k]))
            e = [jnp.exp(a[k] - m) for k in range(4)]
            tot = e[0].sum() + e[1].sum() + e[2].sum() + e[3].sum()
            totv = zero + tot
            p = [e[k] / totv for k in range(4)]
            corr = [p[k] + b[k] for k in range(4)]

            def top1(vecs):
                m1 = jnp.max(vecs[0])
                for k in range(1, 4):
                    m1 = jnp.maximum(m1, jnp.max(vecs[k]))
                i1 = jnp.int32(1 << 20)
                p1 = jnp.float32(0.0)
                for k in range(4):
                    idx = lanes + _L * k
                    i1 = jnp.minimum(
                        i1, jnp.min(jnp.where(vecs[k] == m1, idx, 1 << 20)))
                for k in range(4):
                    idx = lanes + _L * k
                    p1 = p1 + jnp.sum(jnp.where(idx == i1, p[k], 0.0))
                return i1, p1

            i1, p1 = top1(corr)
            corr2 = [jnp.where(lanes + _L * k == i1, _NEG, corr[k])
                     for k in range(4)]
            i2, p2 = top1(corr2)

            denomv = jnp.maximum(zero + (p1 + p2), _NORM_MIN)
            w1 = (zero + p1) / denomv
            w2 = (zero + p2) / denomv

            for k in range(4):
                idx = lanes + _L * k
                cb = (jnp.where(idx == i1, w1, 0.0)
                      + jnp.where(idx == i2, w2, 0.0))
                crow[pl.ds(_L * k, _L)] = cb * mtf
                crow[pl.ds(_EP + _L * k, _L)] = cb * mvf

        for k in list(range(4)) + list(range(8, 12)):
            sl = pl.ds(_L * k, _L)
            pr[sl] = c0[sl] + c1[sl]
        for k in list(range(4, 8)) + list(range(12, 16)):
            pr[pl.ds(_L * k, _L)] = zero

        pltpu.sync_copy(c0, comb_hbm.at[s])
        pltpu.sync_copy(c1, comb_hbm.at[s + _L])
        pltpu.sync_copy(pr, sh.at[s])

    plsc.subcore_barrier()

    def build_schedule(base, out_off):
        pltpu.sync_copy(sh, big)
        carry = jnp.float32(0.0)
        last_e = jnp.int32(0)
        pos = []
        sel = []
        for k in range(4):
            sl = pl.ds(base + _L * k, _L)
            acc = big[0, sl]
            for t in range(1, _L):
                acc = acc + big[t, sl]
            sel_k = acc > 0.0
            idx = lanes + _L * k
            cs = plsc.cumsum(jnp.where(sel_k, 1.0, 0.0)) + carry
            carry = jnp.max(cs)
            pos.append((cs - 1.0).astype(jnp.int32))
            sel.append(sel_k)
            last_e = jnp.maximum(
                last_e, jnp.max(jnp.where(sel_k, idx, -1)))
        last_e = jnp.maximum(last_e, 0)
        for k in range(4):
            schedv[pl.ds(_L * k, _L)] = jnp.zeros((_L,), jnp.int32) + last_e
        for k in range(4):
            idx = lanes + _L * k
            plsc.store_scatter(schedv, [pos[k]], idx, mask=sel[k])
        pltpu.sync_copy(schedv, sched_hbm.at[pl.ds(out_off, _E)])

    @pl.when(jnp.logical_and(c == 0, s == 0))
    def _():
        build_schedule(0, 0)

    @pl.when(jnp.logical_and(c == 0, s == 1))
    def _():
        build_schedule(_EP, _E)


def _sc_routing(lcat, meta):
    fn = pl.kernel(
        _sc_routing_body,
        out_type=[
            jax.ShapeDtypeStruct((_T, 2 * _EP), jnp.float32),
            jax.ShapeDtypeStruct((2 * _E,), jnp.int32),
        ],
        mesh=plsc.VectorSubcoreMesh(core_axis_name="c", subcore_axis_name="s"),
        compiler_params=pltpu.CompilerParams(needs_layout_passes=False),
        scratch_types=[
            pltpu.VMEM((2 * _EP,), jnp.float32),
            pltpu.VMEM((2 * _EP,), jnp.float32),
            pltpu.VMEM((2 * _EP,), jnp.float32),
            pltpu.VMEM((2 * _EP,), jnp.float32),
            pltpu.VMEM((2 * _EP,), jnp.float32),
            pltpu.VMEM((2, _EP), jnp.float32),
            pltpu.VMEM_SHARED((_L, 2 * _EP), jnp.float32),
            pltpu.VMEM((_L, 2 * _EP), jnp.float32),
            pltpu.VMEM((_E,), jnp.int32),
        ],
    )
    return fn(lcat, meta)


_NSH = 4
_T0 = _NSH
_V0 = _NSH + _E


def _mega_body(sched_ref, x_ref, sgw_ref, suw_ref, sdw_ref,
               tg_ref, tu_ref, tdn_ref, vg_ref, vu_ref, vdn_ref,
               comb_ref, out_ref):
    i = pl.program_id(0)
    x = x_ref[...]
    lane = jax.lax.broadcasted_iota(jnp.int32, (_T, 2 * _EP), 1)

    @pl.when(i == 0)
    def _():
        out_ref[...] = jnp.zeros_like(out_ref)

    @pl.when(i < _T0)
    def _():
        g = _dotT(x, sgw_ref[...])
        u = _dotT(x, suw_ref[...])
        h = g * jax.nn.sigmoid(g) * u
        out_ref[...] += _dotT(h, sdw_ref[...])

    def expert(e, g_ref, u_ref, dn_ref, lane_off):
        g = jnp.dot(x, g_ref[0], preferred_element_type=jnp.float32)
        u = jnp.dot(x, u_ref[0], preferred_element_type=jnp.float32)
        h = g * jax.nn.sigmoid(g) * u
        y = jnp.dot(h, dn_ref[0], preferred_element_type=jnp.float32)
        crow = jnp.sum(jnp.where(lane == e + lane_off, comb_ref[...], 0.0),
                       axis=1, keepdims=True)
        out_ref[...] += y * crow

    ti = sched_ref[jnp.clip(i - _T0, 0, _E - 1)]
    tp = sched_ref[jnp.clip(i - _T0 - 1, 0, _E - 1)]
    t_fresh = jnp.logical_and(
        jnp.logical_and(i >= _T0, i < _V0),
        jnp.logical_or(i == _T0, ti != tp))

    @pl.when(t_fresh)
    def _():
        expert(ti, tg_ref, tu_ref, tdn_ref, 0)

    vi = sched_ref[jnp.maximum(i, _V0) - _T0]
    vp = sched_ref[jnp.maximum(i - 1, _V0) - _T0]
    v_fresh = jnp.logical_and(
        i >= _V0, jnp.logical_or(i == _V0, vi != vp))

    @pl.when(v_fresh)
    def _():
        expert(vi, vg_ref, vu_ref, vdn_ref, _EP)


def _mega_call(sched, x, sgw, suw, sdw, t_gate_up, t_down, v_gate_up, v_down,
               comb):
    chunk = _SH // _NSH
    fft, ffv = _FF_TEXT, _FF_VIS
    grid_spec = pltpu.PrefetchScalarGridSpec(
        num_scalar_prefetch=1,
        grid=(_NSH + 2 * _E,),
        in_specs=[
            pl.BlockSpec((_T, _H), lambda i, s: (0, 0)),
            pl.BlockSpec((chunk, _H), lambda i, s: (jnp.minimum(i, _NSH - 1), 0)),
            pl.BlockSpec((chunk, _H), lambda i, s: (jnp.minimum(i, _NSH - 1), 0)),
            pl.BlockSpec((_H, chunk), lambda i, s: (0, jnp.minimum(i, _NSH - 1))),
            pl.BlockSpec((1, _H, fft),
                         lambda i, s: (s[jnp.clip(i - _T0, 0, _E - 1)], 0, 0)),
            pl.BlockSpec((1, _H, fft),
                         lambda i, s: (s[jnp.clip(i - _T0, 0, _E - 1)], 0, 1)),
            pl.BlockSpec((1, fft, _H),
                         lambda i, s: (s[jnp.clip(i - _T0, 0, _E - 1)], 0, 0)),
            pl.BlockSpec((1, _H, ffv),
                         lambda i, s: (s[jnp.maximum(i, _V0) - _T0], 0, 0)),
            pl.BlockSpec((1, _H, ffv),
                         lambda i, s: (s[jnp.maximum(i, _V0) - _T0], 0, 1)),
            pl.BlockSpec((1, ffv, _H),
                         lambda i, s: (s[jnp.maximum(i, _V0) - _T0], 0, 0)),
            pl.BlockSpec((_T, 2 * _EP), lambda i, s: (0, 0)),
        ],
        out_specs=pl.BlockSpec((_T, _H), lambda i, s: (0, 0)),
    )
    return pl.pallas_call(
        _mega_body,
        grid_spec=grid_spec,
        out_shape=jax.ShapeDtypeStruct((_T, _H), jnp.float32),
    )(sched, x, sgw, suw, sdw, t_gate_up, t_gate_up, t_down,
      v_gate_up, v_gate_up, v_down, comb)


def kernel(hidden_states, moe_mm_token_type_ids, text_router_w, text_bias,
           text_gate_up, text_down, vis_router_w, vis_bias, vis_gate_up,
           vis_down, shared_gate_w, shared_up_w, shared_down_w):
    Bv, Sv, D = hidden_states.shape
    x = hidden_states.reshape(-1, D)
    tt1d = moe_mm_token_type_ids.reshape(-1).astype(jnp.int32)

    rl, lcat, meta = pl.pallas_call(
        _logits_body,
        out_shape=[
            jax.ShapeDtypeStruct((_T, _E), jnp.float32),
            jax.ShapeDtypeStruct((_T, 2 * _EP), jnp.float32),
            jax.ShapeDtypeStruct((2, _EP), jnp.float32),
        ],
    )(x, text_router_w, vis_router_w, tt1d.reshape(_T, 1),
      tt1d.reshape(1, _T), text_bias.reshape(1, _E), vis_bias.reshape(1, _E))

    comb, sched = _sc_routing(lcat, meta)

    final = _mega_call(sched, x, shared_gate_w, shared_up_w, shared_down_w,
                       text_gate_up, text_down, vis_gate_up, vis_down, comb)

    return final.reshape(Bv, Sv, D), rl


# speedup vs baseline: 1.0015x; 1.0015x over previous
import functools

import jax
import jax.numpy as jnp
from jax import lax
from jax.experimental import pallas as pl
from jax.experimental.pallas import tpu as pltpu
from jax.experimental.pallas import tpu_sc as plsc

_B, _S = 8, 4
_T = _B * _S
_H = 1024
_E = 64
_EP = 128
_FF_TEXT = 1024
_FF_VIS = 512
_SH = 2048
_NORM_MIN = 1e-12
_NEG = -1e30
_L = 16


def _dotT(a, b):
    return jax.lax.dot_general(a, b, (((1,), (1,)), ((), ())),
                               preferred_element_type=jnp.float32)


def _logits_body(x_ref, tw_ref, vw_ref, ttc_ref, ttr_ref, tb_ref, vb_ref,
                 rl_ref, lcat_ref, meta_ref):
    x = x_ref[...]
    lt = _dotT(x, tw_ref[...])
    lv = _dotT(x, vw_ref[...])
    rl_ref[...] = jnp.where(ttc_ref[...] != 0, lv, lt)
    lcat_ref[...] = jnp.zeros((_T, 2 * _EP), jnp.float32)
    lcat_ref[:, :_E] = lt
    lcat_ref[:, _EP:_EP + _E] = lv
    meta_ref[...] = jnp.zeros((2, _EP), jnp.float32)
    meta_ref[0:1, :_T] = ttr_ref[...].astype(jnp.float32)
    meta_ref[0:1, _T:_T + _E] = tb_ref[...]
    meta_ref[1:2, :_E] = vb_ref[...]


def _sc_routing_body(lcat_hbm, meta_hbm, comb_hbm, sched_hbm,
                     row0, row1, c0, c1, pr, meta, sh, big, schedv):
    c = lax.axis_index("c")
    s = lax.axis_index("s")
    lanes = lax.broadcasted_iota(jnp.int32, (_L,), 0)
    zero = jnp.zeros((_L,), jnp.float32)

    @pl.when(c == 0)
    def _():
        pltpu.sync_copy(meta_hbm, meta)
        pltpu.sync_copy(lcat_hbm.at[s], row0)
        pltpu.sync_copy(lcat_hbm.at[s + _L], row1)

        for crow in (c0, c1):
            for k in list(range(4, 8)) + list(range(12, 16)):
                crow[pl.ds(_L * k, _L)] = zero

        for r, row, crow in ((0, row0, c0), (1, row1, c1)):
            ttf = meta[0, pl.ds(_L * r, _L)]
            tt_t = jnp.sum(jnp.where(lanes == s, ttf, 0.0))
            mtf = jnp.where(tt_t == 0.0, 1.0, 0.0)
            mvf = 1.0 - mtf

            a = []
            b = []
            for k in range(4):
                a.append(row[pl.ds(_L * k, _L)] * mtf
                         + row[pl.ds(_EP + _L * k, _L)] * mvf)
                b.append(meta[0, pl.ds(_T + _L * k, _L)] * mtf
                         + meta[1, pl.ds(_L * k, _L)] * mvf)

            m = jnp.max(a[0])
            for k in range(1, 4):
                m = jnp.maximum(m, jnp.max(a[k]))
            e = [jnp.exp(a[k] - m) for k in range(4)]
            tot = e[0].sum() + e[1].sum() + e[2].sum() + e[3].sum()
            totv = zero + tot
            p = [e[k] / totv for k in range(4)]
            corr = [p[k] + b[k] for k in range(4)]

            def top1(vecs):
                m1 = jnp.max(vecs[0])
                for k in range(1, 4):
                    m1 = jnp.maximum(m1, jnp.max(vecs[k]))
                i1 = jnp.int32(1 << 20)
                p1 = jnp.float32(0.0)
                for k in range(4):
                    idx = lanes + _L * k
                    i1 = jnp.minimum(
                        i1, jnp.min(jnp.where(vecs[k] == m1, idx, 1 << 20)))
                for k in range(4):
                    idx = lanes + _L * k
                    p1 = p1 + jnp.sum(jnp.where(idx == i1, p[k], 0.0))
                return i1, p1

            i1, p1 = top1(corr)
            corr2 = [jnp.where(lanes + _L * k == i1, _NEG, corr[k])
                     for k in range(4)]
            i2, p2 = top1(corr2)

            denomv = jnp.maximum(zero + (p1 + p2), _NORM_MIN)
            w1 = (zero + p1) / denomv
            w2 = (zero + p2) / denomv

            for k in range(4):
                idx = lanes + _L * k
                cb = (jnp.where(idx == i1, w1, 0.0)
                      + jnp.where(idx == i2, w2, 0.0))
                crow[pl.ds(_L * k, _L)] = cb * mtf
                crow[pl.ds(_EP + _L * k, _L)] = cb * mvf

        for k in list(range(4)) + list(range(8, 12)):
            sl = pl.ds(_L * k, _L)
            pr[sl] = c0[sl] + c1[sl]
        for k in list(range(4, 8)) + list(range(12, 16)):
            pr[pl.ds(_L * k, _L)] = zero

        pltpu.sync_copy(c0, comb_hbm.at[s])
        pltpu.sync_copy(c1, comb_hbm.at[s + _L])
        pltpu.sync_copy(pr, sh.at[s])

    plsc.subcore_barrier()

    def build_schedule(base, out_off):
        pltpu.sync_copy(sh, big)
        carry = jnp.float32(0.0)
        last_e = jnp.int32(0)
        pos = []
        sel = []
        for k in range(4):
            sl = pl.ds(base + _L * k, _L)
            acc = big[0, sl]
            for t in range(1, _L):
                acc = acc + big[t, sl]
            sel_k = acc > 0.0
            idx = lanes + _L * k
            cs = plsc.cumsum(jnp.where(sel_k, 1.0, 0.0)) + carry
            carry = jnp.max(cs)
            pos.append((cs - 1.0).astype(jnp.int32))
            sel.append(sel_k)
            last_e = jnp.maximum(
                last_e, jnp.max(jnp.where(sel_k, idx, -1)))
        last_e = jnp.maximum(last_e, 0)
        for k in range(4):
            schedv[pl.ds(_L * k, _L)] = jnp.zeros((_L,), jnp.int32) + last_e
        for k in range(4):
            idx = lanes + _L * k
            plsc.store_scatter(schedv, [pos[k]], idx, mask=sel[k])
        pltpu.sync_copy(schedv, sched_hbm.at[pl.ds(out_off, _E)])

    @pl.when(jnp.logical_and(c == 0, s == 0))
    def _():
        build_schedule(0, 0)

    @pl.when(jnp.logical_and(c == 0, s == 1))
    def _():
        build_schedule(_EP, _E)


def _sc_routing(lcat, meta):
    fn = pl.kernel(
        _sc_routing_body,
        out_type=[
            jax.ShapeDtypeStruct((_T, 2 * _EP), jnp.float32),
            jax.ShapeDtypeStruct((2 * _E,), jnp.int32),
        ],
        mesh=plsc.VectorSubcoreMesh(core_axis_name="c", subcore_axis_name="s"),
        compiler_params=pltpu.CompilerParams(needs_layout_passes=False),
        scratch_types=[
            pltpu.VMEM((2 * _EP,), jnp.float32),
            pltpu.VMEM((2 * _EP,), jnp.float32),
            pltpu.VMEM((2 * _EP,), jnp.float32),
            pltpu.VMEM((2 * _EP,), jnp.float32),
            pltpu.VMEM((2 * _EP,), jnp.float32),
            pltpu.VMEM((2, _EP), jnp.float32),
            pltpu.VMEM_SHARED((_L, 2 * _EP), jnp.float32),
            pltpu.VMEM((_L, 2 * _EP), jnp.float32),
            pltpu.VMEM((_E,), jnp.int32),
        ],
    )
    return fn(lcat, meta)


_NSH = 4
_T0 = _NSH
_V0 = _NSH + _E


def _mega_body(sched_ref, x_ref, sgw_ref, suw_ref, sdw_ref,
               tg_ref, tu_ref, tdn_ref, vg_ref, vu_ref, vdn_ref,
               comb_ref, out_ref):
    i = pl.program_id(0)
    x = x_ref[...]
    lane = jax.lax.broadcasted_iota(jnp.int32, (_T, 2 * _EP), 1)

    @pl.when(i == 0)
    def _():
        out_ref[...] = jnp.zeros_like(out_ref)

    @pl.when(i < _T0)
    def _():
        g = _dotT(x, sgw_ref[...])
        u = _dotT(x, suw_ref[...])
        h = g * jax.nn.sigmoid(g) * u
        out_ref[...] += _dotT(h, sdw_ref[...])

    def expert(e, g_ref, u_ref, dn_ref, lane_off):
        g = jnp.dot(x, g_ref[0], preferred_element_type=jnp.float32)
        u = jnp.dot(x, u_ref[0], preferred_element_type=jnp.float32)
        h = g * jax.nn.sigmoid(g) * u
        y = jnp.dot(h, dn_ref[0], preferred_element_type=jnp.float32)
        crow = jnp.sum(jnp.where(lane == e + lane_off, comb_ref[...], 0.0),
                       axis=1, keepdims=True)
        out_ref[...] += y * crow

    ti = sched_ref[jnp.clip(i - _T0, 0, _E - 1)]
    tp = sched_ref[jnp.clip(i - _T0 - 1, 0, _E - 1)]
    t_fresh = jnp.logical_and(
        jnp.logical_and(i >= _T0, i < _V0),
        jnp.logical_or(i == _T0, ti != tp))

    @pl.when(t_fresh)
    def _():
        expert(ti, tg_ref, tu_ref, tdn_ref, 0)

    vi = sched_ref[jnp.maximum(i, _V0) - _T0]
    vp = sched_ref[jnp.maximum(i - 1, _V0) - _T0]
    v_fresh = jnp.logical_and(
        i >= _V0, jnp.logical_or(i == _V0, vi != vp))

    @pl.when(v_fresh)
    def _():
        expert(vi, vg_ref, vu_ref, vdn_ref, _EP)


def _mega_call(sched, x, sgw, suw, sdw, t_gate_up, t_down, v_gate_up, v_down,
               comb):
    chunk = _SH // _NSH
    fft, ffv = _FF_TEXT, _FF_VIS
    grid_spec = pltpu.PrefetchScalarGridSpec(
        num_scalar_prefetch=1,
        grid=(_NSH + 2 * _E,),
        in_specs=[
            pl.BlockSpec((_T, _H), lambda i, s: (0, 0)),
            pl.BlockSpec((chunk, _H), lambda i, s: (jnp.minimum(i, _NSH - 1), 0)),
            pl.BlockSpec((chunk, _H), lambda i, s: (jnp.minimum(i, _NSH - 1), 0)),
            pl.BlockSpec((_H, chunk), lambda i, s: (0, jnp.minimum(i, _NSH - 1))),
            pl.BlockSpec((1, _H, fft),
                         lambda i, s: (s[jnp.clip(i - _T0, 0, _E - 1)], 0, 0)),
            pl.BlockSpec((1, _H, fft),
                         lambda i, s: (s[jnp.clip(i - _T0, 0, _E - 1)], 0, 1)),
            pl.BlockSpec((1, fft, _H),
                         lambda i, s: (s[jnp.clip(i - _T0, 0, _E - 1)], 0, 0)),
            pl.BlockSpec((1, _H, ffv),
                         lambda i, s: (s[jnp.maximum(i, _V0) - _T0], 0, 0)),
            pl.BlockSpec((1, _H, ffv),
                         lambda i, s: (s[jnp.maximum(i, _V0) - _T0], 0, 1)),
            pl.BlockSpec((1, ffv, _H),
                         lambda i, s: (s[jnp.maximum(i, _V0) - _T0], 0, 0)),
            pl.BlockSpec((_T, 2 * _EP), lambda i, s: (0, 0)),
        ],
        out_specs=pl.BlockSpec((_T, _H), lambda i, s: (0, 0)),
    )
    return pl.pallas_call(
        _mega_body,
        grid_spec=grid_spec,
        compiler_params=pltpu.CompilerParams(
            vmem_limit_bytes=110 * 1024 * 1024),
        out_shape=jax.ShapeDtypeStruct((_T, _H), jnp.float32),
    )(sched, x, sgw, suw, sdw, t_gate_up, t_gate_up, t_down,
      v_gate_up, v_gate_up, v_down, comb)


def kernel(hidden_states, moe_mm_token_type_ids, text_router_w, text_bias,
           text_gate_up, text_down, vis_router_w, vis_bias, vis_gate_up,
           vis_down, shared_gate_w, shared_up_w, shared_down_w):
    Bv, Sv, D = hidden_states.shape
    x = hidden_states.reshape(-1, D)
    tt1d = moe_mm_token_type_ids.reshape(-1).astype(jnp.int32)

    rl, lcat, meta = pl.pallas_call(
        _logits_body,
        out_shape=[
            jax.ShapeDtypeStruct((_T, _E), jnp.float32),
            jax.ShapeDtypeStruct((_T, 2 * _EP), jnp.float32),
            jax.ShapeDtypeStruct((2, _EP), jnp.float32),
        ],
    )(x, text_router_w, vis_router_w, tt1d.reshape(_T, 1),
      tt1d.reshape(1, _T), text_bias.reshape(1, _E), vis_bias.reshape(1, _E))

    comb, sched = _sc_routing(lcat, meta)

    final = _mega_call(sched, x, shared_gate_w, shared_up_w, shared_down_w,
                       text_gate_up, text_down, vis_gate_up, vis_down, comb)

    return final.reshape(Bv, Sv, D), rl


# speedup vs baseline: 1.0542x; 1.0526x over previous
import functools

import jax
import jax.numpy as jnp
from jax import lax
from jax.experimental import pallas as pl
from jax.experimental.pallas import tpu as pltpu
from jax.experimental.pallas import tpu_sc as plsc

_B, _S = 8, 4
_T = _B * _S
_H = 1024
_E = 64
_EP = 128
_FF_TEXT = 1024
_FF_VIS = 512
_SH = 2048
_NORM_MIN = 1e-12
_NEG = -1e30
_L = 16


def _dotT(a, b):
    return jax.lax.dot_general(a, b, (((1,), (1,)), ((), ())),
                               preferred_element_type=jnp.float32)


def _logits_body(x_ref, tw_ref, vw_ref, ttc_ref, ttr_ref, tb_ref, vb_ref,
                 rl_ref, lcat_ref, meta_ref):
    x = x_ref[...]
    lt = _dotT(x, tw_ref[...])
    lv = _dotT(x, vw_ref[...])
    rl_ref[...] = jnp.where(ttc_ref[...] != 0, lv, lt)
    lcat_ref[...] = jnp.zeros((_T, 2 * _EP), jnp.float32)
    lcat_ref[:, :_E] = lt
    lcat_ref[:, _EP:_EP + _E] = lv
    meta_ref[...] = jnp.zeros((2, _EP), jnp.float32)
    meta_ref[0:1, :_T] = ttr_ref[...].astype(jnp.float32)
    meta_ref[0:1, _T:_T + _E] = tb_ref[...]
    meta_ref[1:2, :_E] = vb_ref[...]


def _sc_routing_body(lcat_hbm, meta_hbm, comb_hbm, sched_t_hbm, sched_v_hbm,
                     row0, row1, c0, c1, pr, meta, sh, big, schedv):
    c = lax.axis_index("c")
    s = lax.axis_index("s")
    lanes = lax.broadcasted_iota(jnp.int32, (_L,), 0)
    zero = jnp.zeros((_L,), jnp.float32)

    @pl.when(c == 0)
    def _():
        pltpu.sync_copy(meta_hbm, meta)
        pltpu.sync_copy(lcat_hbm.at[s], row0)
        pltpu.sync_copy(lcat_hbm.at[s + _L], row1)

        for crow in (c0, c1):
            for k in list(range(4, 8)) + list(range(12, 16)):
                crow[pl.ds(_L * k, _L)] = zero

        for r, row, crow in ((0, row0, c0), (1, row1, c1)):
            ttf = meta[0, pl.ds(_L * r, _L)]
            tt_t = jnp.sum(jnp.where(lanes == s, ttf, 0.0))
            mtf = jnp.where(tt_t == 0.0, 1.0, 0.0)
            mvf = 1.0 - mtf

            a = []
            b = []
            for k in range(4):
                a.append(row[pl.ds(_L * k, _L)] * mtf
                         + row[pl.ds(_EP + _L * k, _L)] * mvf)
                b.append(meta[0, pl.ds(_T + _L * k, _L)] * mtf
                         + meta[1, pl.ds(_L * k, _L)] * mvf)

            m = jnp.max(a[0])
            for k in range(1, 4):
                m = jnp.maximum(m, jnp.max(a[k]))
            e = [jnp.exp(a[k] - m) for k in range(4)]
            tot = e[0].sum() + e[1].sum() + e[2].sum() + e[3].sum()
            totv = zero + tot
            p = [e[k] / totv for k in range(4)]
            corr = [p[k] + b[k] for k in range(4)]

            def top1(vecs):
                m1 = jnp.max(vecs[0])
                for k in range(1, 4):
                    m1 = jnp.maximum(m1, jnp.max(vecs[k]))
                i1 = jnp.int32(1 << 20)
                p1 = jnp.float32(0.0)
                for k in range(4):
                    idx = lanes + _L * k
                    i1 = jnp.minimum(
                        i1, jnp.min(jnp.where(vecs[k] == m1, idx, 1 << 20)))
                for k in range(4):
                    idx = lanes + _L * k
                    p1 = p1 + jnp.sum(jnp.where(idx == i1, p[k], 0.0))
                return i1, p1

            i1, p1 = top1(corr)
            corr2 = [jnp.where(lanes + _L * k == i1, _NEG, corr[k])
                     for k in range(4)]
            i2, p2 = top1(corr2)

            denomv = jnp.maximum(zero + (p1 + p2), _NORM_MIN)
            w1 = (zero + p1) / denomv
            w2 = (zero + p2) / denomv

            for k in range(4):
                idx = lanes + _L * k
                cb = (jnp.where(idx == i1, w1, 0.0)
                      + jnp.where(idx == i2, w2, 0.0))
                crow[pl.ds(_L * k, _L)] = cb * mtf
                crow[pl.ds(_EP + _L * k, _L)] = cb * mvf

        for k in list(range(4)) + list(range(8, 12)):
            sl = pl.ds(_L * k, _L)
            pr[sl] = c0[sl] + c1[sl]
        for k in list(range(4, 8)) + list(range(12, 16)):
            pr[pl.ds(_L * k, _L)] = zero

        pltpu.sync_copy(c0, comb_hbm.at[s])
        pltpu.sync_copy(c1, comb_hbm.at[s + _L])
        pltpu.sync_copy(pr, sh.at[s])

    plsc.subcore_barrier()

    def build_schedule(base, sched_hbm):
        pltpu.sync_copy(sh, big)
        carry = jnp.float32(0.0)
        last_e = jnp.int32(0)
        pos = []
        sel = []
        for k in range(4):
            sl = pl.ds(base + _L * k, _L)
            acc = big[0, sl]
            for t in range(1, _L):
                acc = acc + big[t, sl]
            sel_k = acc > 0.0
            idx = lanes + _L * k
            cs = plsc.cumsum(jnp.where(sel_k, 1.0, 0.0)) + carry
            carry = jnp.max(cs)
            pos.append((cs - 1.0).astype(jnp.int32))
            sel.append(sel_k)
            last_e = jnp.maximum(
                last_e, jnp.max(jnp.where(sel_k, idx, -1)))
        last_e = jnp.maximum(last_e, 0)
        for k in range(4):
            schedv[pl.ds(_L * k, _L)] = jnp.zeros((_L,), jnp.int32) + last_e
        for k in range(4):
            idx = lanes + _L * k
            plsc.store_scatter(schedv, [pos[k]], idx, mask=sel[k])
        pltpu.sync_copy(schedv, sched_hbm)

    @pl.when(jnp.logical_and(c == 0, s == 0))
    def _():
        build_schedule(0, sched_t_hbm)

    @pl.when(jnp.logical_and(c == 0, s == 1))
    def _():
        build_schedule(_EP, sched_v_hbm)


def _sc_routing(lcat, meta):
    fn = pl.kernel(
        _sc_routing_body,
        out_type=[
            jax.ShapeDtypeStruct((_T, 2 * _EP), jnp.float32),
            jax.ShapeDtypeStruct((_E,), jnp.int32),
            jax.ShapeDtypeStruct((_E,), jnp.int32),
        ],
        mesh=plsc.VectorSubcoreMesh(core_axis_name="c", subcore_axis_name="s"),
        compiler_params=pltpu.CompilerParams(needs_layout_passes=False),
        scratch_types=[
            pltpu.VMEM((2 * _EP,), jnp.float32),
            pltpu.VMEM((2 * _EP,), jnp.float32),
            pltpu.VMEM((2 * _EP,), jnp.float32),
            pltpu.VMEM((2 * _EP,), jnp.float32),
            pltpu.VMEM((2 * _EP,), jnp.float32),
            pltpu.VMEM((2, _EP), jnp.float32),
            pltpu.VMEM_SHARED((_L, 2 * _EP), jnp.float32),
            pltpu.VMEM((_L, 2 * _EP), jnp.float32),
            pltpu.VMEM((_E,), jnp.int32),
        ],
    )
    return fn(lcat, meta)


def _shared_body(x_ref, gw_ref, uw_ref, dw_ref, out_ref):
    i = pl.program_id(0)

    @pl.when(i == 0)
    def _():
        out_ref[...] = jnp.zeros_like(out_ref)

    x = x_ref[...]
    g = _dotT(x, gw_ref[...])
    u = _dotT(x, uw_ref[...])
    h = g * jax.nn.sigmoid(g) * u
    out_ref[...] += _dotT(h, dw_ref[...])


def _expert_body(sched_ref, x_ref, g_ref, u_ref, dn_ref, comb_ref, base_ref,
                 out_ref, *, lane_off):
    i = pl.program_id(0)
    e = sched_ref[i]
    ep = sched_ref[jnp.maximum(i - 1, 0)]
    fresh = jnp.logical_or(i == 0, e != ep)

    @pl.when(i == 0)
    def _():
        out_ref[...] = base_ref[...]

    @pl.when(fresh)
    def _():
        x = x_ref[...]
        g = jnp.dot(x, g_ref[0], preferred_element_type=jnp.float32)
        u = jnp.dot(x, u_ref[0], preferred_element_type=jnp.float32)
        h = g * jax.nn.sigmoid(g) * u
        y = jnp.dot(h, dn_ref[0], preferred_element_type=jnp.float32)
        lane = jax.lax.broadcasted_iota(jnp.int32, (_T, 2 * _EP), 1)
        crow = jnp.sum(jnp.where(lane == e + lane_off, comb_ref[...], 0.0),
                       axis=1, keepdims=True)
        out_ref[...] += y * crow


def _expert_call(sched, x, gate_up, down, comb, base, ff, lane_off):
    grid_spec = pltpu.PrefetchScalarGridSpec(
        num_scalar_prefetch=1,
        grid=(_E,),
        in_specs=[
            pl.BlockSpec((_T, _H), lambda i, s: (0, 0)),
            pl.BlockSpec((1, _H, ff), lambda i, s: (s[i], 0, 0)),
            pl.BlockSpec((1, _H, ff), lambda i, s: (s[i], 0, 1)),
            pl.BlockSpec((1, ff, _H), lambda i, s: (s[i], 0, 0)),
            pl.BlockSpec((_T, 2 * _EP), lambda i, s: (0, 0)),
            pl.BlockSpec((_T, _H), lambda i, s: (0, 0)),
        ],
        out_specs=pl.BlockSpec((_T, _H), lambda i, s: (0, 0)),
    )
    return pl.pallas_call(
        functools.partial(_expert_body, lane_off=lane_off),
        grid_spec=grid_spec,
        out_shape=jax.ShapeDtypeStruct((_T, _H), jnp.float32),
    )(sched, x, gate_up, gate_up, down, comb, base)


def kernel(hidden_states, moe_mm_token_type_ids, text_router_w, text_bias,
           text_gate_up, text_down, vis_router_w, vis_bias, vis_gate_up,
           vis_down, shared_gate_w, shared_up_w, shared_down_w):
    Bv, Sv, D = hidden_states.shape
    x = hidden_states.reshape(-1, D)
    tt1d = moe_mm_token_type_ids.reshape(-1).astype(jnp.int32)

    rl, lcat, meta = pl.pallas_call(
        _logits_body,
        out_shape=[
            jax.ShapeDtypeStruct((_T, _E), jnp.float32),
            jax.ShapeDtypeStruct((_T, 2 * _EP), jnp.float32),
            jax.ShapeDtypeStruct((2, _EP), jnp.float32),
        ],
    )(x, text_router_w, vis_router_w, tt1d.reshape(_T, 1),
      tt1d.reshape(1, _T), text_bias.reshape(1, _E), vis_bias.reshape(1, _E))

    comb, sched_t, sched_v = _sc_routing(lcat, meta)

    n_chunks = 4
    chunk = _SH // n_chunks
    shared = pl.pallas_call(
        _shared_body,
        grid=(n_chunks,),
        in_specs=[
            pl.BlockSpec((_T, _H), lambda i: (0, 0)),
            pl.BlockSpec((chunk, _H), lambda i: (i, 0)),
            pl.BlockSpec((chunk, _H), lambda i: (i, 0)),
            pl.BlockSpec((_H, chunk), lambda i: (0, i)),
        ],
        out_specs=pl.BlockSpec((_T, _H), lambda i: (0, 0)),
        out_shape=jax.ShapeDtypeStruct((_T, _H), jnp.float32),
    )(x, shared_gate_w, shared_up_w, shared_down_w)

    acc = _expert_call(sched_t, x, text_gate_up, text_down, comb, shared,
                       _FF_TEXT, 0)
    final = _expert_call(sched_v, x, vis_gate_up, vis_down, comb, acc,
                         _FF_VIS, _EP)

    return final.reshape(Bv, Sv, D), rl
